# baseline (speedup 1.0000x reference)
import functools

import jax
import jax.numpy as jnp
from jax import lax
from jax.experimental import pallas as pl
from jax.experimental.pallas import tpu as pltpu
from jax.experimental.pallas import tpu_sc as plsc

N = 4096
D = 512
K = 2048

_NCHUNKS = 4
_KC = K // _NCHUNKS

_NC, _NS = 2, 16
_NW = _NC * _NS
_BPW = _KC // _NW
_CH = 8
_NSUB = _BPW // _CH


def _gather_body(a_hbm, idx_hbm, out_hbm, idx_v, buf0, buf1, gsem, wsem0, wsem1):
    wid = lax.axis_index("s") * _NC + lax.axis_index("c")
    base = wid * _BPW
    pltpu.sync_copy(idx_hbm.at[wid], idx_v)
    bufs = (buf0, buf1)
    wsems = (wsem0, wsem1)
    pending = [None, None]
    for c in range(_NSUB):
        slot = c % 2
        if pending[slot] is not None:
            pending[slot].wait()
        pltpu.async_copy(a_hbm.at[idx_v.at[c]], bufs[slot], gsem).wait()
        pending[slot] = pltpu.async_copy(
            bufs[slot], out_hbm.at[pl.ds(base + c * _CH, _CH)], wsems[slot])
    for p in pending:
        if p is not None:
            p.wait()


def _gather_rows(a, idx3d):
    mesh = plsc.VectorSubcoreMesh(core_axis_name="c", subcore_axis_name="s")
    return pl.kernel(
        _gather_body,
        mesh=mesh,
        out_type=jax.ShapeDtypeStruct((_KC, N), jnp.float32),
        scratch_types=[
            pltpu.VMEM((_NSUB, _CH), jnp.int32),
            pltpu.VMEM((_CH, N), jnp.float32),
            pltpu.VMEM((_CH, N), jnp.float32),
            pltpu.SemaphoreType.DMA,
            pltpu.SemaphoreType.DMA,
            pltpu.SemaphoreType.DMA,
        ],
    )(a, idx3d)


_BM = 256
_GM = _KC // _BM


def _mm_body(ag_ref, x_ref, wt_ref, b_ref, val_ref, out_ref):
    ag_bf = ag_ref[...].astype(jnp.bfloat16)
    acc = jnp.dot(ag_bf, x_ref[...], preferred_element_type=jnp.float32)
    h = jnp.dot(acc.astype(jnp.bfloat16), wt_ref[...],
                preferred_element_type=jnp.float32)
    out_ref[...] = (h + b_ref[...]) * val_ref[...]


def _matmul(ag, x_bf, wt_bf, b2d, val2d):
    return pl.pallas_call(
        _mm_body,
        grid=(_GM,),
        in_specs=[
            pl.BlockSpec((_BM, N), lambda i: (i, 0)),
            pl.BlockSpec((N, D), lambda i: (0, 0)),
            pl.BlockSpec((D, D), lambda i: (0, 0)),
            pl.BlockSpec((1, D), lambda i: (0, 0)),
            pl.BlockSpec((_BM, 1), lambda i: (i, 0)),
        ],
        out_specs=pl.BlockSpec((_BM, D), lambda i: (i, 0)),
        out_shape=jax.ShapeDtypeStruct((_KC, D), jnp.float32),
        compiler_params=pltpu.CompilerParams(
            dimension_semantics=("arbitrary",)),
    )(ag, x_bf, wt_bf, b2d, val2d)


def kernel(A, X, idx, value, W, b):
    idx4d = idx.astype(jnp.int32).reshape(_NCHUNKS, _NW, _NSUB, _CH)
    x_bf = X.astype(jnp.bfloat16)
    wt_bf = W.T.astype(jnp.bfloat16)
    b2d = b.reshape(1, D)
    val2d = value.reshape(K, 1)
    outs = []
    for c in range(_NCHUNKS):
        ag = _gather_rows(A, idx4d[c])
        outs.append(_matmul(ag, x_bf, wt_bf, b2d,
                            lax.slice(val2d, (c * _KC, 0), ((c + 1) * _KC, 1))))
    return jnp.concatenate(outs, axis=0)


# speedup vs baseline: 1.1646x; 1.1646x over previous
import functools

import jax
import jax.numpy as jnp
from jax import lax
from jax.experimental import pallas as pl
from jax.experimental.pallas import tpu as pltpu
from jax.experimental.pallas import tpu_sc as plsc

N = 4096
D = 512
K = 2048

_NCHUNKS = 2
_KC = K // _NCHUNKS

_NC, _NS = 2, 16
_NW = _NC * _NS
_BPW = _KC // _NW
_CH = 8
_NSUB = _BPW // _CH


def _gather_body(a_hbm, idx_hbm, out_hbm, idx_v, buf0, buf1, gsem, wsem0, wsem1):
    wid = lax.axis_index("s") * _NC + lax.axis_index("c")
    base = wid * _BPW
    pltpu.sync_copy(idx_hbm.at[wid], idx_v)
    bufs = (buf0, buf1)
    wsems = (wsem0, wsem1)
    pending = [None, None]
    for c in range(_NSUB):
        slot = c % 2
        if pending[slot] is not None:
            pending[slot].wait()
        pltpu.async_copy(a_hbm.at[idx_v.at[c]], bufs[slot], gsem).wait()
        pending[slot] = pltpu.async_copy(
            bufs[slot], out_hbm.at[pl.ds(base + c * _CH, _CH)], wsems[slot])
    for p in pending:
        if p is not None:
            p.wait()


def _gather_rows(a, idx3d):
    mesh = plsc.VectorSubcoreMesh(core_axis_name="c", subcore_axis_name="s")
    return pl.kernel(
        _gather_body,
        mesh=mesh,
        out_type=jax.ShapeDtypeStruct((_KC, N), jnp.float32),
        scratch_types=[
            pltpu.VMEM((_NSUB, _CH), jnp.int32),
            pltpu.VMEM((_CH, N), jnp.float32),
            pltpu.VMEM((_CH, N), jnp.float32),
            pltpu.SemaphoreType.DMA,
            pltpu.SemaphoreType.DMA,
            pltpu.SemaphoreType.DMA,
        ],
    )(a, idx3d)


_BM = 256
_GM = _KC // _BM


def _mm_body(ag_ref, x_ref, wt_ref, b_ref, val_ref, prev_ref, out_ref):
    del prev_ref
    ag_bf = ag_ref[...].astype(jnp.bfloat16)
    acc = jnp.dot(ag_bf, x_ref[...], preferred_element_type=jnp.float32)
    h = jnp.dot(acc.astype(jnp.bfloat16), wt_ref[...],
                preferred_element_type=jnp.float32)
    out_ref[...] = (h + b_ref[...]) * val_ref[...]


def _matmul_chunk(chunk, ag, x_bf, wt_bf, b2d, val2d, prev):
    off = chunk * _GM
    return pl.pallas_call(
        _mm_body,
        grid=(_GM,),
        in_specs=[
            pl.BlockSpec((_BM, N), lambda i: (i, 0)),
            pl.BlockSpec((N, D), lambda i: (0, 0)),
            pl.BlockSpec((D, D), lambda i: (0, 0)),
            pl.BlockSpec((1, D), lambda i: (0, 0)),
            pl.BlockSpec((_BM, 1), lambda i: (off + i, 0)),
            pl.BlockSpec(memory_space=pl.ANY),
        ],
        out_specs=pl.BlockSpec((_BM, D), lambda i: (off + i, 0)),
        out_shape=jax.ShapeDtypeStruct((K, D), jnp.float32),
        input_output_aliases={5: 0},
        compiler_params=pltpu.CompilerParams(
            dimension_semantics=("arbitrary",)),
    )(ag, x_bf, wt_bf, b2d, val2d, prev)


def kernel(A, X, idx, value, W, b):
    idx4d = idx.astype(jnp.int32).reshape(_NCHUNKS, _NW, _NSUB, _CH)
    x_bf = X.astype(jnp.bfloat16)
    wt_bf = W.T.astype(jnp.bfloat16)
    b2d = b.reshape(1, D)
    val2d = value.reshape(K, 1)
    ags = [_gather_rows(A, idx4d[c]) for c in range(_NCHUNKS)]
    out = jnp.zeros((K, D), jnp.float32)
    for c in range(_NCHUNKS):
        out = _matmul_chunk(c, ags[c], x_bf, wt_bf, b2d, val2d, out)
    return out


# speedup vs baseline: 1.1816x; 1.0146x over previous
import functools

import jax
import jax.numpy as jnp
from jax import lax
from jax.experimental import pallas as pl
from jax.experimental.pallas import tpu as pltpu
from jax.experimental.pallas import tpu_sc as plsc

N = 4096
D = 512
K = 2048

_NCHUNKS = 2
_KC = K // _NCHUNKS

_NC, _NS = 2, 16
_NW = _NC * _NS
_BPW = _KC // _NW
_CH = 8
_NSUB = _BPW // _CH


def _gather_body(a_hbm, idx_hbm, out_hbm, idx_v, buf0, buf1,
                 gsem0, gsem1, wsem0, wsem1, *, chunk_off):
    wid = lax.axis_index("s") * _NC + lax.axis_index("c")
    base = wid * _BPW
    pltpu.sync_copy(idx_hbm.at[pl.ds(chunk_off + base, _BPW)], idx_v)
    bufs = (buf0, buf1)
    gsems = (gsem0, gsem1)
    wsems = (wsem0, wsem1)
    gpend = [None, None]
    wpend = [None, None]
    gpend[0] = pltpu.async_copy(
        a_hbm.at[idx_v.at[pl.ds(0, _CH)]], bufs[0], gsems[0])
    for c in range(_NSUB):
        slot = c % 2
        nxt = (c + 1) % 2
        gpend[slot].wait()
        if c + 1 < _NSUB:
            if wpend[nxt] is not None:
                wpend[nxt].wait()
                wpend[nxt] = None
            gpend[nxt] = pltpu.async_copy(
                a_hbm.at[idx_v.at[pl.ds((c + 1) * _CH, _CH)]],
                bufs[nxt], gsems[nxt])
        wpend[slot] = pltpu.async_copy(
            bufs[slot], out_hbm.at[pl.ds(base + c * _CH, _CH)], wsems[slot])
    for p in wpend:
        if p is not None:
            p.wait()


def _gather_rows(a, idx, chunk):
    mesh = plsc.VectorSubcoreMesh(core_axis_name="c", subcore_axis_name="s")
    return pl.kernel(
        functools.partial(_gather_body, chunk_off=chunk * _KC),
        mesh=mesh,
        out_type=jax.ShapeDtypeStruct((_KC, N), jnp.float32),
        scratch_types=[
            pltpu.VMEM((_BPW,), jnp.int32),
            pltpu.VMEM((_CH, N), jnp.float32),
            pltpu.VMEM((_CH, N), jnp.float32),
            pltpu.SemaphoreType.DMA,
            pltpu.SemaphoreType.DMA,
            pltpu.SemaphoreType.DMA,
            pltpu.SemaphoreType.DMA,
        ],
    )(a, idx)


_BM = 256
_GM = _KC // _BM


def _mm_body(ag_ref, x_ref, wt_ref, b_ref, val_ref, *rest):
    out_ref = rest[-1]
    ag_bf = ag_ref[...].astype(jnp.bfloat16)
    acc = jnp.dot(ag_bf, x_ref[...], preferred_element_type=jnp.float32)
    h = jnp.dot(acc.astype(jnp.bfloat16), wt_ref[...],
                preferred_element_type=jnp.float32)
    out_ref[...] = (h + b_ref[...]) * val_ref[...]


def _matmul_chunk(chunk, ag, x_bf, wt_bf, b2d, val2d, prev):
    off = chunk * _GM
    in_specs = [
        pl.BlockSpec((_BM, N), lambda i: (i, 0)),
        pl.BlockSpec((N, D), lambda i: (0, 0)),
        pl.BlockSpec((D, D), lambda i: (0, 0)),
        pl.BlockSpec((1, D), lambda i: (0, 0)),
        pl.BlockSpec((_BM, 1), lambda i: (off + i, 0)),
    ]
    args = [ag, x_bf, wt_bf, b2d, val2d]
    aliases = {}
    if prev is not None:
        in_specs.append(pl.BlockSpec(memory_space=pl.ANY))
        args.append(prev)
        aliases = {5: 0}
    return pl.pallas_call(
        _mm_body,
        grid=(_GM,),
        in_specs=in_specs,
        out_specs=pl.BlockSpec((_BM, D), lambda i: (off + i, 0)),
        out_shape=jax.ShapeDtypeStruct((K, D), jnp.float32),
        input_output_aliases=aliases,
        compiler_params=pltpu.CompilerParams(
            dimension_semantics=("arbitrary",)),
    )(*args)


def kernel(A, X, idx, value, W, b):
    idx32 = idx.astype(jnp.int32)
    x_bf = X.astype(jnp.bfloat16)
    wt_bf = W.T.astype(jnp.bfloat16)
    b2d = b.reshape(1, D)
    val2d = value.reshape(K, 1)
    ags = [_gather_rows(A, idx32, c) for c in range(_NCHUNKS)]
    out = None
    for c in range(_NCHUNKS):
        out = _matmul_chunk(c, ags[c], x_bf, wt_bf, b2d, val2d, out)
    return out
